# baseline (speedup 1.0000x reference)
import functools

import jax
import jax.numpy as jnp
from jax import lax
from jax.experimental import pallas as pl
from jax.experimental.pallas import tpu as pltpu
from jax.experimental.pallas import tpu_sc as plsc

_NC = 2
_NS = 16
_NW = _NC * _NS
_L = 16


def _cat_body(*refs, bpw, num_fields):
  idx_hbm = refs[0]
  tbl_refs = refs[1:1 + num_fields]
  out_hbm = refs[1 + num_fields]
  rest = refs[2 + num_fields:]
  idx_v, out_v = rest[:2]
  fidx_bufs = rest[2:2 + num_fields]
  g_bufs = rest[2 + num_fields:2 + 2 * num_fields]
  sem = rest[2 + 2 * num_fields]

  wid = lax.axis_index("s") * _NC + lax.axis_index("c")
  pltpu.sync_copy(idx_hbm.at[pl.ds(wid * bpw * num_fields, bpw * num_fields)],
                  idx_v)

  iota = lax.iota(jnp.int32, _L)
  iota_f = iota * num_fields
  nchunks = bpw // _L

  for f in range(num_fields):
    def j_body(j, _, f=f):
      fidx_bufs[f][pl.ds(j * _L, _L)] = plsc.load_gather(
          idx_v, [iota_f + (j * _L * num_fields + f)])
      return 0

    lax.fori_loop(0, nchunks, j_body, 0)
    pltpu.async_copy(tbl_refs[f].at[fidx_bufs[f]], g_bufs[f], sem)

  for f in range(num_fields):
    pltpu.make_async_copy(
        tbl_refs[f].at[fidx_bufs[f]], g_bufs[f], sem).wait()

  def chunk_body(j, _):
    sl = pl.ds(j * _L, _L)
    acc = g_bufs[0][sl]
    for f in range(1, num_fields):
      acc = acc + g_bufs[f][sl]
    out_v[sl] = acc
    return 0

  lax.fori_loop(0, nchunks, chunk_body, 0)
  pltpu.sync_copy(out_v, out_hbm.at[pl.ds(wid * bpw, bpw)])


def _lin_body(cat_ref, xn_ref, w_ref, b_ref, o_ref):
  num = jax.lax.dot_general(
      xn_ref[...], w_ref[...], (((1,), (1,)), ((), ())),
      preferred_element_type=jnp.float32)
  o_ref[...] = cat_ref[...][:, None] + num + b_ref[0]


@functools.partial(jax.jit, static_argnames=())
def kernel(x_cat, x_num, tables, W, bias):
  B, F = x_cat.shape
  _, D_NUM = x_num.shape
  bpw = B // _NW

  idx = x_cat.reshape(B * F)
  tbl_slices = [tables[f, :, 0] for f in range(F)]

  mesh = plsc.VectorSubcoreMesh(core_axis_name="c", subcore_axis_name="s",
                                num_cores=_NC, num_subcores=_NS)
  cat_body = functools.partial(_cat_body, bpw=bpw, num_fields=F)
  cat = pl.kernel(
      cat_body,
      out_type=jax.ShapeDtypeStruct((B,), jnp.float32),
      mesh=mesh,
      compiler_params=pltpu.CompilerParams(needs_layout_passes=False),
      scratch_types=(
          [pltpu.VMEM((bpw * F,), jnp.int32),
           pltpu.VMEM((bpw,), jnp.float32)]
          + [pltpu.VMEM((bpw,), jnp.int32) for _ in range(F)]
          + [pltpu.VMEM((bpw,), jnp.float32) for _ in range(F)]
          + [pltpu.SemaphoreType.DMA]
      ),
  )(idx, *tbl_slices)

  out = pl.pallas_call(
      _lin_body,
      out_shape=jax.ShapeDtypeStruct((B, 1), jnp.float32),
  )(cat, x_num, W, bias)
  return out
